# Initial kernel scaffold; baseline (speedup 1.0000x reference)
import jax
import jax.numpy as jnp
from jax.experimental import pallas as pl


def kernel(state, W1, b1, W2, b2, W3, b3, EW1, Eb1, EW2, Eb2, EW3, Eb3):
    raise NotImplementedError("write your pallas kernel here")


# speedup vs baseline: 1.3363x; 1.3363x over previous
import functools

import jax
import jax.numpy as jnp
from jax import lax
from jax.experimental import pallas as pl
from jax.experimental.pallas import tpu as pltpu

TOK = 4096
SD = 1024
HID = 1024
ADIM = 512
NE = 8
BLK = 128
NBLK = TOK // BLK + NE
PAD = NBLK * BLK
RB = 512


def _router_body(g_ref, x_ref, w1_ref, b1_ref, w2_ref, b2_ref, w3_ref, b3_ref,
                 probs_ref, sel_ref):
    x = x_ref[...]
    h = jnp.maximum(x @ w1_ref[...] + b1_ref[...], 0.0)
    h = jnp.maximum(h @ w2_ref[...] + b2_ref[...], 0.0)
    logits = h @ w3_ref[...] + b3_ref[...]
    m = jnp.max(logits, axis=-1, keepdims=True)
    p = jnp.exp(logits - m)
    probs_ref[...] = p / jnp.sum(p, axis=-1, keepdims=True)
    z = g_ref[...] + logits
    best = z[:, 0]
    bi = jnp.zeros((RB,), jnp.int32)
    for e in range(1, NE):
        upd = z[:, e] > best
        best = jnp.where(upd, z[:, e], best)
        bi = jnp.where(upd, e, bi)
    sel_ref[0, 0, :] = bi


def _router(state, W1, b1, W2, b2, W3, b3, gumbel):
    nb = TOK // RB
    probs, sel3 = pl.pallas_call(
        _router_body,
        grid=(nb,),
        in_specs=[
            pl.BlockSpec((RB, NE), lambda j: (j, 0)),
            pl.BlockSpec((RB, SD), lambda j: (j, 0)),
            pl.BlockSpec((SD, HID), lambda j: (0, 0)),
            pl.BlockSpec((HID,), lambda j: (0,)),
            pl.BlockSpec((HID, HID), lambda j: (0, 0)),
            pl.BlockSpec((HID,), lambda j: (0,)),
            pl.BlockSpec((HID, NE), lambda j: (0, 0)),
            pl.BlockSpec((NE,), lambda j: (0,)),
        ],
        out_specs=[
            pl.BlockSpec((RB, NE), lambda j: (j, 0)),
            pl.BlockSpec((1, 1, RB), lambda j: (j, 0, 0)),
        ],
        out_shape=[
            jax.ShapeDtypeStruct((TOK, NE), jnp.float32),
            jax.ShapeDtypeStruct((nb, 1, RB), jnp.int32),
        ],
    )(gumbel, state, W1, b1, W2, b2, W3, b3)
    return probs, sel3.reshape(TOK)


def _expert_body(be_ref, x_ref, w1_ref, b1_ref, w2_ref, b2_ref, w3_ref, b3_ref,
                 o_ref):
    x = x_ref[...]
    h = jnp.maximum(x @ w1_ref[0] + b1_ref[0, 0], 0.0)
    h = jnp.maximum(h @ w2_ref[0] + b2_ref[0, 0], 0.0)
    y = h @ w3_ref[0] + b3_ref[0, 0]
    o_ref[...] = jnp.concatenate(
        [y[:, :ADIM], jnp.exp(y[:, ADIM:])], axis=-1)


def _experts(block_expert, xg, EW1, Eb1, EW2, Eb2, EW3, Eb3):
    grid_spec = pltpu.PrefetchScalarGridSpec(
        num_scalar_prefetch=1,
        grid=(NBLK,),
        in_specs=[
            pl.BlockSpec((BLK, SD), lambda j, be: (j, 0)),
            pl.BlockSpec((1, SD, HID), lambda j, be: (be[j], 0, 0)),
            pl.BlockSpec((1, 1, HID), lambda j, be: (be[j], 0, 0)),
            pl.BlockSpec((1, HID, HID), lambda j, be: (be[j], 0, 0)),
            pl.BlockSpec((1, 1, HID), lambda j, be: (be[j], 0, 0)),
            pl.BlockSpec((1, HID, 2 * ADIM), lambda j, be: (be[j], 0, 0)),
            pl.BlockSpec((1, 1, 2 * ADIM), lambda j, be: (be[j], 0, 0)),
        ],
        out_specs=pl.BlockSpec((BLK, 2 * ADIM), lambda j, be: (j, 0)),
    )
    return pl.pallas_call(
        _expert_body,
        grid_spec=grid_spec,
        out_shape=jax.ShapeDtypeStruct((PAD, 2 * ADIM), jnp.float32),
    )(block_expert, xg, EW1, Eb1.reshape(NE, 1, HID), EW2,
      Eb2.reshape(NE, 1, HID), EW3, Eb3.reshape(NE, 1, 2 * ADIM))


def _routing_indices(sel):
    counts = jnp.bincount(sel, length=NE)
    order = jnp.argsort(sel)
    pblocks = (counts + BLK - 1) // BLK
    cum_blocks = jnp.cumsum(pblocks)
    pstart = jnp.concatenate([jnp.zeros(1, jnp.int32),
                              cum_blocks[:-1].astype(jnp.int32)]) * BLK
    offs = jnp.concatenate([jnp.zeros(1, jnp.int32),
                            jnp.cumsum(counts)[:-1].astype(jnp.int32)])
    e_sorted = sel[order]
    k = jnp.arange(TOK, dtype=jnp.int32)
    slots = pstart[e_sorted] + (k - offs[e_sorted])
    slot_tok = jnp.full((PAD,), -1, jnp.int32).at[slots].set(order.astype(jnp.int32))
    is_pad = slot_tok < 0
    pad_rank = jnp.cumsum(is_pad.astype(jnp.int32)) - 1
    gather_idx = jnp.maximum(slot_tok, 0)
    scatter_idx = jnp.where(is_pad, TOK + pad_rank, slot_tok)
    j = jnp.arange(NBLK, dtype=jnp.int32)
    block_expert = (j[:, None] >= cum_blocks[None, :]).sum(-1).astype(jnp.int32)
    block_expert = jnp.minimum(block_expert, NE - 1)
    return gather_idx, scatter_idx, block_expert


def kernel(state, W1, b1, W2, b2, W3, b3, EW1, Eb1, EW2, Eb2, EW3, Eb3):
    gumbel = jax.random.gumbel(jax.random.key(42), (TOK, NE), jnp.float32)
    probs, sel = _router(state, W1, b1, W2, b2, W3, b3, gumbel)
    gather_idx, scatter_idx, block_expert = _routing_indices(sel)
    xg = state[gather_idx]
    y = _experts(block_expert, xg, EW1, Eb1, EW2, Eb2, EW3, Eb3)
    out = jnp.zeros((PAD, 2 * ADIM), jnp.float32).at[scatter_idx].set(y)
    action_mean = out[:TOK, :ADIM]
    action_std = out[:TOK, ADIM:]
    return probs, sel, action_mean, action_std


# speedup vs baseline: 1.3700x; 1.0252x over previous
import functools

import jax
import jax.numpy as jnp
from jax import lax
from jax.experimental import pallas as pl
from jax.experimental.pallas import tpu as pltpu
from jax.experimental.pallas import tpu_sc as plsc

TOK = 4096
SD = 1024
HID = 1024
ADIM = 512
NE = 8
BLK = 128
NBLK = TOK // BLK + NE
PAD = NBLK * BLK
RB = 512


def _router_body(g_ref, x_ref, w1_ref, b1_ref, w2_ref, b2_ref, w3_ref, b3_ref,
                 probs_ref, sel_ref):
    x = x_ref[...]
    h = jnp.maximum(x @ w1_ref[...] + b1_ref[...], 0.0)
    h = jnp.maximum(h @ w2_ref[...] + b2_ref[...], 0.0)
    logits = h @ w3_ref[...] + b3_ref[...]
    m = jnp.max(logits, axis=-1, keepdims=True)
    p = jnp.exp(logits - m)
    probs_ref[...] = p / jnp.sum(p, axis=-1, keepdims=True)
    z = g_ref[...] + logits
    best = z[:, 0]
    bi = jnp.zeros((RB,), jnp.int32)
    for e in range(1, NE):
        upd = z[:, e] > best
        best = jnp.where(upd, z[:, e], best)
        bi = jnp.where(upd, e, bi)
    sel_ref[0, 0, :] = bi


def _router(state, W1, b1, W2, b2, W3, b3, gumbel):
    nb = TOK // RB
    probs, sel3 = pl.pallas_call(
        _router_body,
        grid=(nb,),
        in_specs=[
            pl.BlockSpec((RB, NE), lambda j: (j, 0)),
            pl.BlockSpec((RB, SD), lambda j: (j, 0)),
            pl.BlockSpec((SD, HID), lambda j: (0, 0)),
            pl.BlockSpec((HID,), lambda j: (0,)),
            pl.BlockSpec((HID, HID), lambda j: (0, 0)),
            pl.BlockSpec((HID,), lambda j: (0,)),
            pl.BlockSpec((HID, NE), lambda j: (0, 0)),
            pl.BlockSpec((NE,), lambda j: (0,)),
        ],
        out_specs=[
            pl.BlockSpec((RB, NE), lambda j: (j, 0)),
            pl.BlockSpec((1, 1, RB), lambda j: (j, 0, 0)),
        ],
        out_shape=[
            jax.ShapeDtypeStruct((TOK, NE), jnp.float32),
            jax.ShapeDtypeStruct((nb, 1, RB), jnp.int32),
        ],
    )(gumbel, state, W1, b1, W2, b2, W3, b3)
    return probs, sel3.reshape(TOK)


def _expert_body(be_ref, x_ref, w1_ref, b1_ref, w2_ref, b2_ref, w3_ref, b3_ref,
                 o_ref):
    x = x_ref[...]
    h = jnp.maximum(x @ w1_ref[0] + b1_ref[0, 0], 0.0)
    h = jnp.maximum(h @ w2_ref[0] + b2_ref[0, 0], 0.0)
    y = h @ w3_ref[0] + b3_ref[0, 0]
    o_ref[...] = jnp.concatenate(
        [y[:, :ADIM], jnp.exp(y[:, ADIM:])], axis=-1)


def _experts(block_expert, xg, EW1, Eb1, EW2, Eb2, EW3, Eb3):
    grid_spec = pltpu.PrefetchScalarGridSpec(
        num_scalar_prefetch=1,
        grid=(NBLK,),
        in_specs=[
            pl.BlockSpec((BLK, SD), lambda j, be: (j, 0)),
            pl.BlockSpec((1, SD, HID), lambda j, be: (be[j], 0, 0)),
            pl.BlockSpec((1, 1, HID), lambda j, be: (be[j], 0, 0)),
            pl.BlockSpec((1, HID, HID), lambda j, be: (be[j], 0, 0)),
            pl.BlockSpec((1, 1, HID), lambda j, be: (be[j], 0, 0)),
            pl.BlockSpec((1, HID, 2 * ADIM), lambda j, be: (be[j], 0, 0)),
            pl.BlockSpec((1, 1, 2 * ADIM), lambda j, be: (be[j], 0, 0)),
        ],
        out_specs=pl.BlockSpec((BLK, 2 * ADIM), lambda j, be: (j, 0)),
    )
    return pl.pallas_call(
        _expert_body,
        grid_spec=grid_spec,
        out_shape=jax.ShapeDtypeStruct((PAD, 2 * ADIM), jnp.float32),
    )(block_expert, xg, EW1, Eb1.reshape(NE, 1, HID), EW2,
      Eb2.reshape(NE, 1, HID), EW3, Eb3.reshape(NE, 1, 2 * ADIM))


_NW = 32
_RPW = PAD // _NW
_CH = 80
_NCH = _RPW // _CH


def _sc_mesh():
    return plsc.VectorSubcoreMesh(core_axis_name="c", subcore_axis_name="s")


def _sc_gather(state, gather_idx):
    @functools.partial(
        pl.kernel, mesh=_sc_mesh(),
        out_type=jax.ShapeDtypeStruct((PAD, SD), jnp.float32),
        scratch_types=[pltpu.VMEM((_CH,), jnp.int32),
                       pltpu.VMEM((_CH, SD), jnp.float32),
                       pltpu.SemaphoreType.DMA])
    def k(state_hbm, idx_hbm, out_hbm, idx_v, rows_v, sem):
        wid = lax.axis_index("s") * 2 + lax.axis_index("c")
        base = wid * _RPW
        for c in range(_NCH):
            off = base + c * _CH
            pltpu.sync_copy(idx_hbm.at[pl.ds(off, _CH)], idx_v)
            pltpu.async_copy(state_hbm.at[idx_v], rows_v, sem).wait()
            pltpu.sync_copy(rows_v, out_hbm.at[pl.ds(off, _CH)])
    return k(state, gather_idx)


def _sc_scatter(y, scatter_idx):
    @functools.partial(
        pl.kernel, mesh=_sc_mesh(),
        out_type=jax.ShapeDtypeStruct((PAD, 2 * ADIM), jnp.float32),
        scratch_types=[pltpu.VMEM((_NCH, _CH), jnp.int32),
                       pltpu.VMEM((_CH, 2 * ADIM), jnp.float32),
                       pltpu.SemaphoreType.DMA])
    def k(y_hbm, idx_hbm, out_hbm, idx_v, rows_v, sem):
        wid = lax.axis_index("s") * 2 + lax.axis_index("c")
        base = wid * _RPW
        pltpu.sync_copy(idx_hbm.at[wid], idx_v)
        for c in range(_NCH):
            pltpu.sync_copy(y_hbm.at[pl.ds(base + c * _CH, _CH)], rows_v)
            pltpu.async_copy(rows_v, out_hbm.at[idx_v.at[c]], sem).wait()
    return k(y, scatter_idx.reshape(_NW, _NCH, _CH))


def _routing_indices(sel):
    counts = jnp.bincount(sel, length=NE)
    order = jnp.argsort(sel)
    pblocks = (counts + BLK - 1) // BLK
    cum_blocks = jnp.cumsum(pblocks)
    pstart = jnp.concatenate([jnp.zeros(1, jnp.int32),
                              cum_blocks[:-1].astype(jnp.int32)]) * BLK
    offs = jnp.concatenate([jnp.zeros(1, jnp.int32),
                            jnp.cumsum(counts)[:-1].astype(jnp.int32)])
    e_sorted = sel[order]
    k = jnp.arange(TOK, dtype=jnp.int32)
    slots = pstart[e_sorted] + (k - offs[e_sorted])
    slot_tok = jnp.full((PAD,), -1, jnp.int32).at[slots].set(order.astype(jnp.int32))
    is_pad = slot_tok < 0
    pad_rank = jnp.cumsum(is_pad.astype(jnp.int32)) - 1
    gather_idx = jnp.maximum(slot_tok, 0)
    scatter_idx = jnp.where(is_pad, TOK + pad_rank, slot_tok)
    j = jnp.arange(NBLK, dtype=jnp.int32)
    block_expert = (j[:, None] >= cum_blocks[None, :]).sum(-1).astype(jnp.int32)
    block_expert = jnp.minimum(block_expert, NE - 1)
    return gather_idx, scatter_idx, block_expert


def kernel(state, W1, b1, W2, b2, W3, b3, EW1, Eb1, EW2, Eb2, EW3, Eb3):
    gumbel = jax.random.gumbel(jax.random.key(42), (TOK, NE), jnp.float32)
    probs, sel = _router(state, W1, b1, W2, b2, W3, b3, gumbel)
    gather_idx, scatter_idx, block_expert = _routing_indices(sel)
    xg = _sc_gather(state, gather_idx)
    y = _experts(block_expert, xg, EW1, Eb1, EW2, Eb2, EW3, Eb3)
    out = _sc_scatter(y, scatter_idx)
    action_mean = out[:TOK, :ADIM]
    action_std = out[:TOK, ADIM:]
    return probs, sel, action_mean, action_std


# speedup vs baseline: 1.5878x; 1.1589x over previous
import functools

import jax
import jax.numpy as jnp
from jax import lax
from jax.experimental import pallas as pl
from jax.experimental.pallas import tpu as pltpu
from jax.experimental.pallas import tpu_sc as plsc

TOK = 4096
SD = 1024
HID = 1024
ADIM = 512
NE = 8
BLK = 128
NBLK = TOK // BLK + NE
PAD = NBLK * BLK
RB = 512


def _router_body(g_ref, x_ref, w1_ref, b1_ref, w2_ref, b2_ref, w3_ref, b3_ref,
                 probs_ref, sel_ref):
    x = x_ref[...]
    h = jnp.maximum(x @ w1_ref[...] + b1_ref[...], 0.0)
    h = jnp.maximum(h @ w2_ref[...] + b2_ref[...], 0.0)
    logits = h @ w3_ref[...] + b3_ref[...]
    m = jnp.max(logits, axis=-1, keepdims=True)
    p = jnp.exp(logits - m)
    probs_ref[...] = p / jnp.sum(p, axis=-1, keepdims=True)
    z = g_ref[...] + logits
    best = z[:, 0]
    bi = jnp.zeros((RB,), jnp.int32)
    for e in range(1, NE):
        upd = z[:, e] > best
        best = jnp.where(upd, z[:, e], best)
        bi = jnp.where(upd, e, bi)
    sel_ref[0, 0, :] = bi


def _router(state, W1, b1, W2, b2, W3, b3, gumbel):
    nb = TOK // RB
    probs, sel3 = pl.pallas_call(
        _router_body,
        grid=(nb,),
        in_specs=[
            pl.BlockSpec((RB, NE), lambda j: (j, 0)),
            pl.BlockSpec((RB, SD), lambda j: (j, 0)),
            pl.BlockSpec((SD, HID), lambda j: (0, 0)),
            pl.BlockSpec((HID,), lambda j: (0,)),
            pl.BlockSpec((HID, HID), lambda j: (0, 0)),
            pl.BlockSpec((HID,), lambda j: (0,)),
            pl.BlockSpec((HID, NE), lambda j: (0, 0)),
            pl.BlockSpec((NE,), lambda j: (0,)),
        ],
        out_specs=[
            pl.BlockSpec((RB, NE), lambda j: (j, 0)),
            pl.BlockSpec((1, 1, RB), lambda j: (j, 0, 0)),
        ],
        out_shape=[
            jax.ShapeDtypeStruct((TOK, NE), jnp.float32),
            jax.ShapeDtypeStruct((nb, 1, RB), jnp.int32),
        ],
    )(gumbel, state, W1, b1, W2, b2, W3, b3)
    return probs, sel3.reshape(TOK)


def _expert_body(be_ref, x_ref, w1_ref, b1_ref, w2_ref, b2_ref, w3_ref, b3_ref,
                 o_ref):
    x = x_ref[...]
    h = jnp.maximum(x @ w1_ref[0] + b1_ref[0, 0], 0.0)
    h = jnp.maximum(h @ w2_ref[0] + b2_ref[0, 0], 0.0)
    y = h @ w3_ref[0] + b3_ref[0, 0]
    o_ref[...] = jnp.concatenate(
        [y[:, :ADIM], jnp.exp(y[:, ADIM:])], axis=-1)


def _experts(block_expert, xg, EW1, Eb1, EW2, Eb2, EW3, Eb3):
    grid_spec = pltpu.PrefetchScalarGridSpec(
        num_scalar_prefetch=1,
        grid=(NBLK,),
        in_specs=[
            pl.BlockSpec((BLK, SD), lambda j, be: (j, 0)),
            pl.BlockSpec((1, SD, HID), lambda j, be: (be[j], 0, 0)),
            pl.BlockSpec((1, 1, HID), lambda j, be: (be[j], 0, 0)),
            pl.BlockSpec((1, HID, HID), lambda j, be: (be[j], 0, 0)),
            pl.BlockSpec((1, 1, HID), lambda j, be: (be[j], 0, 0)),
            pl.BlockSpec((1, HID, 2 * ADIM), lambda j, be: (be[j], 0, 0)),
            pl.BlockSpec((1, 1, 2 * ADIM), lambda j, be: (be[j], 0, 0)),
        ],
        out_specs=pl.BlockSpec((BLK, 2 * ADIM), lambda j, be: (j, 0)),
    )
    return pl.pallas_call(
        _expert_body,
        grid_spec=grid_spec,
        out_shape=jax.ShapeDtypeStruct((PAD, 2 * ADIM), jnp.float32),
    )(block_expert, xg, EW1, Eb1.reshape(NE, 1, HID), EW2,
      Eb2.reshape(NE, 1, HID), EW3, Eb3.reshape(NE, 1, 2 * ADIM))


_NW = 32
_RPW = PAD // _NW
_CH = 80
_NCH = _RPW // _CH


def _sc_mesh():
    return plsc.VectorSubcoreMesh(core_axis_name="c", subcore_axis_name="s")


def _sc_gather(state, gather_idx):
    @functools.partial(
        pl.kernel, mesh=_sc_mesh(),
        out_type=jax.ShapeDtypeStruct((PAD, SD), jnp.float32),
        scratch_types=[pltpu.VMEM((_CH,), jnp.int32),
                       pltpu.VMEM((_CH, SD), jnp.float32),
                       pltpu.SemaphoreType.DMA])
    def k(state_hbm, idx_hbm, out_hbm, idx_v, rows_v, sem):
        wid = lax.axis_index("s") * 2 + lax.axis_index("c")
        base = wid * _RPW
        for c in range(_NCH):
            off = base + c * _CH
            pltpu.sync_copy(idx_hbm.at[pl.ds(off, _CH)], idx_v)
            pltpu.async_copy(state_hbm.at[idx_v], rows_v, sem).wait()
            pltpu.sync_copy(rows_v, out_hbm.at[pl.ds(off, _CH)])
    return k(state, gather_idx)


def _sc_scatter(y, scatter_idx):
    @functools.partial(
        pl.kernel, mesh=_sc_mesh(),
        out_type=jax.ShapeDtypeStruct((TOK + PAD, 2 * ADIM), jnp.float32),
        scratch_types=[pltpu.VMEM((_NCH, _CH), jnp.int32),
                       pltpu.VMEM((_CH, 2 * ADIM), jnp.float32),
                       pltpu.SemaphoreType.DMA])
    def k(y_hbm, idx_hbm, out_hbm, idx_v, rows_v, sem):
        wid = lax.axis_index("s") * 2 + lax.axis_index("c")
        base = wid * _RPW
        pltpu.sync_copy(idx_hbm.at[wid], idx_v)
        for c in range(_NCH):
            pltpu.sync_copy(y_hbm.at[pl.ds(base + c * _CH, _CH)], rows_v)
            pltpu.async_copy(rows_v, out_hbm.at[idx_v.at[c]], sem).wait()
    return k(y, scatter_idx.reshape(_NW, _NCH, _CH))


_RNW = 16
_RT = TOK // _RNW
_STRIPE = PAD // _RNW


def _sc_routing(sel):
    @functools.partial(
        pl.kernel,
        mesh=plsc.VectorSubcoreMesh(core_axis_name="c", subcore_axis_name="s",
                                    num_cores=1),
        out_type=[jax.ShapeDtypeStruct((PAD,), jnp.int32),
                  jax.ShapeDtypeStruct((PAD,), jnp.int32),
                  jax.ShapeDtypeStruct((48,), jnp.int32)],
        compiler_params=pltpu.CompilerParams(needs_layout_passes=False),
        scratch_types=[
            pltpu.VMEM((_RT,), jnp.int32),
            pltpu.VMEM((_RT,), jnp.int32),
            pltpu.VMEM((16,), jnp.int32),
            pltpu.VMEM((_RNW * 16,), jnp.int32),
            pltpu.VMEM((_STRIPE,), jnp.int32),
            pltpu.VMEM((_STRIPE,), jnp.int32),
            pltpu.VMEM((_STRIPE,), jnp.int32),
            pltpu.VMEM((48,), jnp.int32),
            pltpu.VMEM_SHARED((_RNW * 16,), jnp.int32),
            pltpu.VMEM_SHARED((PAD,), jnp.int32),
        ])
    def k(sel_hbm, gidx_hbm, sidx_hbm, be_hbm,
          sel_v, tok1_v, cnt_row_v, cnt_all_v, stripe_v,
          gidx_v, sidx_v, be_v, counts_sh, slot_sh):
        w = lax.axis_index("s")
        iota = lax.iota(jnp.int32, 16)
        base_tok = w * _RT
        pltpu.sync_copy(sel_hbm.at[pl.ds(base_tok, _RT)], sel_v)

        def splat_sum(v):
            top = jnp.where(iota == 0, jnp.flip(plsc.cumsum(v)), 0)
            return plsc.cumsum(top)

        cnts = [jnp.zeros((16,), jnp.int32) for _ in range(NE)]
        toks_by_step = []
        for t in range(16):
            tok = plsc.load_gather(sel_v, [iota * 16 + t])
            toks_by_step.append(tok)
            for e in range(NE):
                cnts[e] = cnts[e] + jnp.where(tok == e, 1, 0)
        wcnt = jnp.zeros((16,), jnp.int32)
        prefs = []
        for e in range(NE):
            wcnt = wcnt + jnp.where(iota == e, splat_sum(cnts[e]), 0)
            prefs.append(plsc.cumsum(cnts[e]) - cnts[e])
        cnt_row_v[...] = wcnt
        pltpu.sync_copy(cnt_row_v, counts_sh.at[pl.ds(w * 16, 16)])
        for c in range(_STRIPE // 16):
            stripe_v[pl.ds(c * 16, 16)] = jnp.full((16,), -1, jnp.int32)
        pltpu.sync_copy(stripe_v, slot_sh.at[pl.ds(w * _STRIPE, _STRIPE)])
        plsc.subcore_barrier()

        pltpu.sync_copy(counts_sh, cnt_all_v)
        total = jnp.zeros((16,), jnp.int32)
        prior = jnp.zeros((16,), jnp.int32)
        for t in range(_RNW):
            row = cnt_all_v[pl.ds(t * 16, 16)]
            total = total + row
            prior = jnp.where(jnp.full((16,), t, jnp.int32) < w,
                              prior + row, prior)
        pblk = (total + (BLK - 1)) >> 7
        ptok = pblk << 7
        cum_ptok = plsc.cumsum(ptok)
        pstart = cum_ptok - ptok
        cum_blk = plsc.cumsum(pblk)
        base_vec = pstart + prior
        base_splats = [splat_sum(jnp.where(iota == e, base_vec, 0))
                       for e in range(NE)]

        runs = [jnp.zeros((16,), jnp.int32) for _ in range(NE)]
        poss = []
        for t in range(16):
            tok = toks_by_step[t]
            pos = jnp.zeros((16,), jnp.int32)
            for e in range(NE):
                m = tok == e
                pos = jnp.where(m, base_splats[e] + prefs[e] + runs[e], pos)
                runs[e] = runs[e] + jnp.where(m, 1, 0)
            poss.append(pos)
            tok1_v[pl.ds(t * 16, 16)] = base_tok + iota * 16 + t
        for t in range(16):
            pltpu.sync_copy(tok1_v.at[pl.ds(t * 16, 16)],
                            slot_sh.at[poss[t]])
        plsc.subcore_barrier()

        pltpu.sync_copy(slot_sh.at[pl.ds(w * _STRIPE, _STRIPE)], stripe_v)
        for c in range(_STRIPE // 16):
            st = stripe_v[pl.ds(c * 16, 16)]
            slot_pos = w * _STRIPE + c * 16 + iota
            gidx_v[pl.ds(c * 16, 16)] = jnp.maximum(st, 0)
            sidx_v[pl.ds(c * 16, 16)] = jnp.where(st < 0, TOK + slot_pos, st)
        pltpu.sync_copy(gidx_v, gidx_hbm.at[pl.ds(w * _STRIPE, _STRIPE)])
        pltpu.sync_copy(sidx_v, sidx_hbm.at[pl.ds(w * _STRIPE, _STRIPE)])

        @pl.when(w == 0)
        def _():
            cb_splats = [splat_sum(jnp.where(iota == e, cum_blk, 0))
                         for e in range(NE)]
            for c in range(3):
                j = c * 16 + iota
                be = jnp.zeros((16,), jnp.int32)
                for e in range(NE):
                    be = be + jnp.where(j >= cb_splats[e], 1, 0)
                be_v[pl.ds(c * 16, 16)] = jnp.minimum(be, NE - 1)
            pltpu.sync_copy(be_v, be_hbm)

    return k(sel)


def kernel(state, W1, b1, W2, b2, W3, b3, EW1, Eb1, EW2, Eb2, EW3, Eb3):
    gumbel = jax.random.gumbel(jax.random.key(42), (TOK, NE), jnp.float32)
    probs, sel = _router(state, W1, b1, W2, b2, W3, b3, gumbel)
    gather_idx, scatter_idx, block_expert = _sc_routing(sel)
    xg = _sc_gather(state, gather_idx)
    y = _experts(block_expert, xg, EW1, Eb1, EW2, Eb2, EW3, Eb3)
    out = _sc_scatter(y, scatter_idx)
    action_mean = out[:TOK, :ADIM]
    action_std = out[:TOK, ADIM:]
    return probs, sel, action_mean, action_std


# speedup vs baseline: 1.5966x; 1.0055x over previous
import functools

import jax
import jax.numpy as jnp
from jax import lax
from jax.experimental import pallas as pl
from jax.experimental.pallas import tpu as pltpu
from jax.experimental.pallas import tpu_sc as plsc

TOK = 4096
SD = 1024
HID = 1024
ADIM = 512
NE = 8
BLK = 128
NBLK = TOK // BLK + NE
PAD = NBLK * BLK
RB = 512


def _router_body(g_ref, x_ref, w1_ref, b1_ref, w2_ref, b2_ref, w3_ref, b3_ref,
                 probs_ref, sel_ref):
    x = x_ref[...]
    h = jnp.maximum(x @ w1_ref[...] + b1_ref[...], 0.0)
    h = jnp.maximum(h @ w2_ref[...] + b2_ref[...], 0.0)
    logits = h @ w3_ref[...] + b3_ref[...]
    m = jnp.max(logits, axis=-1, keepdims=True)
    p = jnp.exp(logits - m)
    probs_ref[...] = p / jnp.sum(p, axis=-1, keepdims=True)
    z = g_ref[...] + logits
    best = z[:, 0]
    bi = jnp.zeros((RB,), jnp.int32)
    for e in range(1, NE):
        upd = z[:, e] > best
        best = jnp.where(upd, z[:, e], best)
        bi = jnp.where(upd, e, bi)
    sel_ref[0, 0, :] = bi


def _router(state, W1, b1, W2, b2, W3, b3, gumbel):
    nb = TOK // RB
    probs, sel3 = pl.pallas_call(
        _router_body,
        grid=(nb,),
        in_specs=[
            pl.BlockSpec((RB, NE), lambda j: (j, 0)),
            pl.BlockSpec((RB, SD), lambda j: (j, 0)),
            pl.BlockSpec((SD, HID), lambda j: (0, 0)),
            pl.BlockSpec((HID,), lambda j: (0,)),
            pl.BlockSpec((HID, HID), lambda j: (0, 0)),
            pl.BlockSpec((HID,), lambda j: (0,)),
            pl.BlockSpec((HID, NE), lambda j: (0, 0)),
            pl.BlockSpec((NE,), lambda j: (0,)),
        ],
        out_specs=[
            pl.BlockSpec((RB, NE), lambda j: (j, 0)),
            pl.BlockSpec((1, 1, RB), lambda j: (j, 0, 0)),
        ],
        out_shape=[
            jax.ShapeDtypeStruct((TOK, NE), jnp.float32),
            jax.ShapeDtypeStruct((nb, 1, RB), jnp.int32),
        ],
    )(gumbel, state, W1, b1, W2, b2, W3, b3)
    return probs, sel3.reshape(TOK)


def _expert_body(be_ref, x_ref, w1_ref, b1_ref, w2_ref, b2_ref, w3_ref, b3_ref,
                 o_ref):
    x = x_ref[...]
    h = jnp.maximum(x @ w1_ref[0] + b1_ref[0, 0], 0.0)
    h = jnp.maximum(h @ w2_ref[0] + b2_ref[0, 0], 0.0)
    y = h @ w3_ref[0] + b3_ref[0, 0]
    o_ref[...] = jnp.concatenate(
        [y[:, :ADIM], jnp.exp(y[:, ADIM:])], axis=-1)


def _experts(block_expert, xg, EW1, Eb1, EW2, Eb2, EW3, Eb3):
    grid_spec = pltpu.PrefetchScalarGridSpec(
        num_scalar_prefetch=1,
        grid=(NBLK,),
        in_specs=[
            pl.BlockSpec((BLK, SD), lambda j, be: (j, 0)),
            pl.BlockSpec((1, SD, HID), lambda j, be: (be[j], 0, 0)),
            pl.BlockSpec((1, 1, HID), lambda j, be: (be[j], 0, 0)),
            pl.BlockSpec((1, HID, HID), lambda j, be: (be[j], 0, 0)),
            pl.BlockSpec((1, 1, HID), lambda j, be: (be[j], 0, 0)),
            pl.BlockSpec((1, HID, 2 * ADIM), lambda j, be: (be[j], 0, 0)),
            pl.BlockSpec((1, 1, 2 * ADIM), lambda j, be: (be[j], 0, 0)),
        ],
        out_specs=pl.BlockSpec((BLK, 2 * ADIM), lambda j, be: (j, 0)),
    )
    return pl.pallas_call(
        _expert_body,
        grid_spec=grid_spec,
        out_shape=jax.ShapeDtypeStruct((PAD, 2 * ADIM), jnp.float32),
    )(block_expert, xg, EW1, Eb1.reshape(NE, 1, HID), EW2,
      Eb2.reshape(NE, 1, HID), EW3, Eb3.reshape(NE, 1, 2 * ADIM))


_NW = 32
_RPW = PAD // _NW
_CH = 80
_NCH = _RPW // _CH


def _sc_mesh():
    return plsc.VectorSubcoreMesh(core_axis_name="c", subcore_axis_name="s")


_GCH = 40
_GN = _RPW // _GCH


def _sc_gather(state, gather_idx):
    @functools.partial(
        pl.kernel, mesh=_sc_mesh(),
        out_type=jax.ShapeDtypeStruct((PAD, SD), jnp.float32),
        scratch_types=[pltpu.VMEM((_RPW,), jnp.int32),
                       pltpu.VMEM((_GCH, SD), jnp.float32),
                       pltpu.VMEM((_GCH, SD), jnp.float32),
                       pltpu.SemaphoreType.DMA, pltpu.SemaphoreType.DMA,
                       pltpu.SemaphoreType.DMA, pltpu.SemaphoreType.DMA])
    def k(state_hbm, idx_hbm, out_hbm, idx_v, buf0, buf1, gs0, gs1, ws0, ws1):
        wid = lax.axis_index("s") * 2 + lax.axis_index("c")
        base = wid * _RPW
        bufs, gsem, wsem = [buf0, buf1], [gs0, gs1], [ws0, ws1]
        pltpu.sync_copy(idx_hbm.at[pl.ds(base, _RPW)], idx_v)
        g = [None] * _GN
        wr = [None] * _GN
        for c in range(2):
            g[c] = pltpu.async_copy(
                state_hbm.at[idx_v.at[pl.ds(c * _GCH, _GCH)]],
                bufs[c % 2], gsem[c % 2])
        for c in range(_GN):
            g[c].wait()
            wr[c] = pltpu.async_copy(
                bufs[c % 2], out_hbm.at[pl.ds(base + c * _GCH, _GCH)],
                wsem[c % 2])
            if c + 2 < _GN:
                wr[c].wait()
                g[c + 2] = pltpu.async_copy(
                    state_hbm.at[idx_v.at[pl.ds((c + 2) * _GCH, _GCH)]],
                    bufs[c % 2], gsem[c % 2])
        for c in range(_GN - 2, _GN):
            wr[c].wait()
    return k(state, gather_idx)


def _sc_scatter(y, scatter_idx):
    @functools.partial(
        pl.kernel, mesh=_sc_mesh(),
        out_type=jax.ShapeDtypeStruct((TOK + PAD, 2 * ADIM), jnp.float32),
        scratch_types=[pltpu.VMEM((_NCH, _CH), jnp.int32),
                       pltpu.VMEM((_CH, 2 * ADIM), jnp.float32),
                       pltpu.SemaphoreType.DMA])
    def k(y_hbm, idx_hbm, out_hbm, idx_v, rows_v, sem):
        wid = lax.axis_index("s") * 2 + lax.axis_index("c")
        base = wid * _RPW
        pltpu.sync_copy(idx_hbm.at[wid], idx_v)
        for c in range(_NCH):
            pltpu.sync_copy(y_hbm.at[pl.ds(base + c * _CH, _CH)], rows_v)
            pltpu.async_copy(rows_v, out_hbm.at[idx_v.at[c]], sem).wait()
    return k(y, scatter_idx.reshape(_NW, _NCH, _CH))


_RNW = 16
_RT = TOK // _RNW
_STRIPE = PAD // _RNW


def _sc_routing(sel):
    @functools.partial(
        pl.kernel,
        mesh=plsc.VectorSubcoreMesh(core_axis_name="c", subcore_axis_name="s",
                                    num_cores=1),
        out_type=[jax.ShapeDtypeStruct((PAD,), jnp.int32),
                  jax.ShapeDtypeStruct((PAD,), jnp.int32),
                  jax.ShapeDtypeStruct((48,), jnp.int32)],
        compiler_params=pltpu.CompilerParams(needs_layout_passes=False),
        scratch_types=[
            pltpu.VMEM((_RT,), jnp.int32),
            pltpu.VMEM((_RT,), jnp.int32),
            pltpu.VMEM((16,), jnp.int32),
            pltpu.VMEM((_RNW * 16,), jnp.int32),
            pltpu.VMEM((_STRIPE,), jnp.int32),
            pltpu.VMEM((_STRIPE,), jnp.int32),
            pltpu.VMEM((_STRIPE,), jnp.int32),
            pltpu.VMEM((48,), jnp.int32),
            pltpu.VMEM_SHARED((_RNW * 16,), jnp.int32),
            pltpu.VMEM_SHARED((PAD,), jnp.int32),
        ])
    def k(sel_hbm, gidx_hbm, sidx_hbm, be_hbm,
          sel_v, tok1_v, cnt_row_v, cnt_all_v, stripe_v,
          gidx_v, sidx_v, be_v, counts_sh, slot_sh):
        w = lax.axis_index("s")
        iota = lax.iota(jnp.int32, 16)
        base_tok = w * _RT
        pltpu.sync_copy(sel_hbm.at[pl.ds(base_tok, _RT)], sel_v)

        def splat_sum(v):
            top = jnp.where(iota == 0, jnp.flip(plsc.cumsum(v)), 0)
            return plsc.cumsum(top)

        cnts = [jnp.zeros((16,), jnp.int32) for _ in range(NE)]
        toks_by_step = []
        for t in range(16):
            tok = plsc.load_gather(sel_v, [iota * 16 + t])
            toks_by_step.append(tok)
            for e in range(NE):
                cnts[e] = cnts[e] + jnp.where(tok == e, 1, 0)
        wcnt = jnp.zeros((16,), jnp.int32)
        prefs = []
        for e in range(NE):
            wcnt = wcnt + jnp.where(iota == e, splat_sum(cnts[e]), 0)
            prefs.append(plsc.cumsum(cnts[e]) - cnts[e])
        cnt_row_v[...] = wcnt
        pltpu.sync_copy(cnt_row_v, counts_sh.at[pl.ds(w * 16, 16)])
        for c in range(_STRIPE // 16):
            stripe_v[pl.ds(c * 16, 16)] = jnp.full((16,), -1, jnp.int32)
        pltpu.sync_copy(stripe_v, slot_sh.at[pl.ds(w * _STRIPE, _STRIPE)])
        plsc.subcore_barrier()

        pltpu.sync_copy(counts_sh, cnt_all_v)
        total = jnp.zeros((16,), jnp.int32)
        prior = jnp.zeros((16,), jnp.int32)
        for t in range(_RNW):
            row = cnt_all_v[pl.ds(t * 16, 16)]
            total = total + row
            prior = jnp.where(jnp.full((16,), t, jnp.int32) < w,
                              prior + row, prior)
        pblk = (total + (BLK - 1)) >> 7
        ptok = pblk << 7
        cum_ptok = plsc.cumsum(ptok)
        pstart = cum_ptok - ptok
        cum_blk = plsc.cumsum(pblk)
        base_vec = pstart + prior
        base_splats = [splat_sum(jnp.where(iota == e, base_vec, 0))
                       for e in range(NE)]

        runs = [jnp.zeros((16,), jnp.int32) for _ in range(NE)]
        poss = []
        for t in range(16):
            tok = toks_by_step[t]
            pos = jnp.zeros((16,), jnp.int32)
            for e in range(NE):
                m = tok == e
                pos = jnp.where(m, base_splats[e] + prefs[e] + runs[e], pos)
                runs[e] = runs[e] + jnp.where(m, 1, 0)
            poss.append(pos)
            tok1_v[pl.ds(t * 16, 16)] = base_tok + iota * 16 + t
        for t in range(16):
            pltpu.sync_copy(tok1_v.at[pl.ds(t * 16, 16)],
                            slot_sh.at[poss[t]])
        plsc.subcore_barrier()

        pltpu.sync_copy(slot_sh.at[pl.ds(w * _STRIPE, _STRIPE)], stripe_v)
        for c in range(_STRIPE // 16):
            st = stripe_v[pl.ds(c * 16, 16)]
            slot_pos = w * _STRIPE + c * 16 + iota
            gidx_v[pl.ds(c * 16, 16)] = jnp.maximum(st, 0)
            sidx_v[pl.ds(c * 16, 16)] = jnp.where(st < 0, TOK + slot_pos, st)
        pltpu.sync_copy(gidx_v, gidx_hbm.at[pl.ds(w * _STRIPE, _STRIPE)])
        pltpu.sync_copy(sidx_v, sidx_hbm.at[pl.ds(w * _STRIPE, _STRIPE)])

        @pl.when(w == 0)
        def _():
            cb_splats = [splat_sum(jnp.where(iota == e, cum_blk, 0))
                         for e in range(NE)]
            for c in range(3):
                j = c * 16 + iota
                be = jnp.zeros((16,), jnp.int32)
                for e in range(NE):
                    be = be + jnp.where(j >= cb_splats[e], 1, 0)
                be_v[pl.ds(c * 16, 16)] = jnp.minimum(be, NE - 1)
            pltpu.sync_copy(be_v, be_hbm)

    return k(sel)


def kernel(state, W1, b1, W2, b2, W3, b3, EW1, Eb1, EW2, Eb2, EW3, Eb3):
    gumbel = jax.random.gumbel(jax.random.key(42), (TOK, NE), jnp.float32)
    probs, sel = _router(state, W1, b1, W2, b2, W3, b3, gumbel)
    gather_idx, scatter_idx, block_expert = _sc_routing(sel)
    xg = _sc_gather(state, gather_idx)
    y = _experts(block_expert, xg, EW1, Eb1, EW2, Eb2, EW3, Eb3)
    out = _sc_scatter(y, scatter_idx)
    action_mean = out[:TOK, :ADIM]
    action_std = out[:TOK, ADIM:]
    return probs, sel, action_mean, action_std


# speedup vs baseline: 2.0492x; 1.2835x over previous
import functools

import jax
import jax.numpy as jnp
from jax import lax
from jax.experimental import pallas as pl
from jax.experimental.pallas import tpu as pltpu
from jax.experimental.pallas import tpu_sc as plsc

TOK = 4096
SD = 1024
HID = 1024
ADIM = 512
NE = 8
BLK = 128
NBLK = TOK // BLK + NE
PAD = NBLK * BLK
RB = 512


def _router_body(g_ref, x_ref, w1_ref, b1_ref, w2_ref, b2_ref, w3_ref, b3_ref,
                 probs_ref, sel_ref):
    x = x_ref[...]
    h = jnp.maximum(x @ w1_ref[...] + b1_ref[...], 0.0)
    h = jnp.maximum(h @ w2_ref[...] + b2_ref[...], 0.0)
    logits = h @ w3_ref[...] + b3_ref[...]
    m = jnp.max(logits, axis=-1, keepdims=True)
    p = jnp.exp(logits - m)
    probs_ref[...] = p / jnp.sum(p, axis=-1, keepdims=True)
    z = g_ref[...] + logits
    best = z[:, 0]
    bi = jnp.zeros((RB,), jnp.int32)
    for e in range(1, NE):
        upd = z[:, e] > best
        best = jnp.where(upd, z[:, e], best)
        bi = jnp.where(upd, e, bi)
    sel_ref[0, 0, :] = bi


def _router(state, W1, b1, W2, b2, W3, b3, gumbel):
    nb = TOK // RB
    probs, sel3 = pl.pallas_call(
        _router_body,
        grid=(nb,),
        in_specs=[
            pl.BlockSpec((RB, NE), lambda j: (j, 0)),
            pl.BlockSpec((RB, SD), lambda j: (j, 0)),
            pl.BlockSpec((SD, HID), lambda j: (0, 0)),
            pl.BlockSpec((HID,), lambda j: (0,)),
            pl.BlockSpec((HID, HID), lambda j: (0, 0)),
            pl.BlockSpec((HID,), lambda j: (0,)),
            pl.BlockSpec((HID, NE), lambda j: (0, 0)),
            pl.BlockSpec((NE,), lambda j: (0,)),
        ],
        out_specs=[
            pl.BlockSpec((RB, NE), lambda j: (j, 0)),
            pl.BlockSpec((1, 1, RB), lambda j: (j, 0, 0)),
        ],
        out_shape=[
            jax.ShapeDtypeStruct((TOK, NE), jnp.float32),
            jax.ShapeDtypeStruct((nb, 1, RB), jnp.int32),
        ],
    )(gumbel, state, W1, b1, W2, b2, W3, b3)
    return probs, sel3.reshape(TOK)


def _expert_body(be_ref, x_ref, w1_ref, b1_ref, w2_ref, b2_ref, w3_ref, b3_ref,
                 o_ref):
    x = x_ref[...]
    h = jnp.maximum(x @ w1_ref[0] + b1_ref[0, 0], 0.0)
    h = jnp.maximum(h @ w2_ref[0] + b2_ref[0, 0], 0.0)
    y = h @ w3_ref[0] + b3_ref[0, 0]
    o_ref[...] = jnp.concatenate(
        [y[:, :ADIM], jnp.exp(y[:, ADIM:])], axis=-1)


def _experts(block_expert, xg, EW1, Eb1, EW2, Eb2, EW3, Eb3):
    grid_spec = pltpu.PrefetchScalarGridSpec(
        num_scalar_prefetch=1,
        grid=(NBLK,),
        in_specs=[
            pl.BlockSpec((BLK, SD), lambda j, be: (j, 0)),
            pl.BlockSpec((1, SD, HID), lambda j, be: (be[j], 0, 0)),
            pl.BlockSpec((1, 1, HID), lambda j, be: (be[j], 0, 0)),
            pl.BlockSpec((1, HID, HID), lambda j, be: (be[j], 0, 0)),
            pl.BlockSpec((1, 1, HID), lambda j, be: (be[j], 0, 0)),
            pl.BlockSpec((1, HID, 2 * ADIM), lambda j, be: (be[j], 0, 0)),
            pl.BlockSpec((1, 1, 2 * ADIM), lambda j, be: (be[j], 0, 0)),
        ],
        out_specs=pl.BlockSpec((BLK, 2 * ADIM), lambda j, be: (j, 0)),
    )
    return pl.pallas_call(
        _expert_body,
        grid_spec=grid_spec,
        out_shape=jax.ShapeDtypeStruct((PAD, 2 * ADIM), jnp.float32),
    )(block_expert, xg, EW1, Eb1.reshape(NE, 1, HID), EW2,
      Eb2.reshape(NE, 1, HID), EW3, Eb3.reshape(NE, 1, 2 * ADIM))


_NW = 32
_RPW = PAD // _NW
_CH = 80
_NCH = _RPW // _CH


def _sc_mesh():
    return plsc.VectorSubcoreMesh(core_axis_name="c", subcore_axis_name="s")


_TPW = TOK // _NW
_SCH = 64


def _sc_shuffle_in(state, pos):
    @functools.partial(
        pl.kernel, mesh=_sc_mesh(),
        out_type=jax.ShapeDtypeStruct((PAD, SD), jnp.float32),
        scratch_types=[pltpu.VMEM((_TPW // _SCH, _SCH), jnp.int32),
                       pltpu.VMEM((_SCH, SD), jnp.float32),
                       pltpu.SemaphoreType.DMA])
    def k(state_hbm, pos_hbm, xg_hbm, idx_v, rows_v, sem):
        wid = lax.axis_index("s") * 2 + lax.axis_index("c")
        base = wid * _TPW
        pltpu.sync_copy(pos_hbm.at[wid], idx_v)
        for c in range(_TPW // _SCH):
            pltpu.sync_copy(state_hbm.at[pl.ds(base + c * _SCH, _SCH)],
                            rows_v)
            pltpu.async_copy(rows_v, xg_hbm.at[idx_v.at[c]], sem).wait()
    return k(state, pos.reshape(_NW, _TPW // _SCH, _SCH))


def _sc_scatter(y, scatter_idx):
    @functools.partial(
        pl.kernel, mesh=_sc_mesh(),
        out_type=jax.ShapeDtypeStruct((TOK + PAD, 2 * ADIM), jnp.float32),
        scratch_types=[pltpu.VMEM((_NCH, _CH), jnp.int32),
                       pltpu.VMEM((_CH, 2 * ADIM), jnp.float32),
                       pltpu.SemaphoreType.DMA])
    def k(y_hbm, idx_hbm, out_hbm, idx_v, rows_v, sem):
        wid = lax.axis_index("s") * 2 + lax.axis_index("c")
        base = wid * _RPW
        pltpu.sync_copy(idx_hbm.at[wid], idx_v)
        for c in range(_NCH):
            pltpu.sync_copy(y_hbm.at[pl.ds(base + c * _CH, _CH)], rows_v)
            pltpu.async_copy(rows_v, out_hbm.at[idx_v.at[c]], sem).wait()
    return k(y, scatter_idx.reshape(_NW, _NCH, _CH))


_RNW = 16
_RT = TOK // _RNW
_STRIPE = PAD // _RNW


def _sc_routing(sel):
    @functools.partial(
        pl.kernel,
        mesh=plsc.VectorSubcoreMesh(core_axis_name="c", subcore_axis_name="s",
                                    num_cores=1),
        out_type=[jax.ShapeDtypeStruct((TOK,), jnp.int32),
                  jax.ShapeDtypeStruct((PAD,), jnp.int32),
                  jax.ShapeDtypeStruct((48,), jnp.int32)],
        compiler_params=pltpu.CompilerParams(needs_layout_passes=False),
        scratch_types=[
            pltpu.VMEM((_RT,), jnp.int32),
            pltpu.VMEM((_RT,), jnp.int32),
            pltpu.VMEM((_RT,), jnp.int32),
            pltpu.VMEM((16,), jnp.int32),
            pltpu.VMEM((_RNW * 16,), jnp.int32),
            pltpu.VMEM((_STRIPE,), jnp.int32),
            pltpu.VMEM((_STRIPE,), jnp.int32),
            pltpu.VMEM((48,), jnp.int32),
            pltpu.VMEM_SHARED((_RNW * 16,), jnp.int32),
            pltpu.VMEM_SHARED((PAD,), jnp.int32),
        ])
    def k(sel_hbm, pos_hbm, sidx_hbm, be_hbm,
          sel_v, tok1_v, pos1_v, cnt_row_v, cnt_all_v, stripe_v,
          sidx_v, be_v, counts_sh, slot_sh):
        w = lax.axis_index("s")
        iota = lax.iota(jnp.int32, 16)
        base_tok = w * _RT
        pltpu.sync_copy(sel_hbm.at[pl.ds(base_tok, _RT)], sel_v)

        def splat_sum(v):
            top = jnp.where(iota == 0, jnp.flip(plsc.cumsum(v)), 0)
            return plsc.cumsum(top)

        cnts = [jnp.zeros((16,), jnp.int32) for _ in range(NE)]
        toks_by_step = []
        for t in range(16):
            tok = plsc.load_gather(sel_v, [iota * 16 + t])
            toks_by_step.append(tok)
            for e in range(NE):
                cnts[e] = cnts[e] + jnp.where(tok == e, 1, 0)
        wcnt = jnp.zeros((16,), jnp.int32)
        prefs = []
        for e in range(NE):
            wcnt = wcnt + jnp.where(iota == e, splat_sum(cnts[e]), 0)
            prefs.append(plsc.cumsum(cnts[e]) - cnts[e])
        cnt_row_v[...] = wcnt
        pltpu.sync_copy(cnt_row_v, counts_sh.at[pl.ds(w * 16, 16)])
        for c in range(_STRIPE // 16):
            stripe_v[pl.ds(c * 16, 16)] = jnp.full((16,), -1, jnp.int32)
        pltpu.sync_copy(stripe_v, slot_sh.at[pl.ds(w * _STRIPE, _STRIPE)])
        plsc.subcore_barrier()

        pltpu.sync_copy(counts_sh, cnt_all_v)
        total = jnp.zeros((16,), jnp.int32)
        prior = jnp.zeros((16,), jnp.int32)
        for t in range(_RNW):
            row = cnt_all_v[pl.ds(t * 16, 16)]
            total = total + row
            prior = jnp.where(jnp.full((16,), t, jnp.int32) < w,
                              prior + row, prior)
        pblk = (total + (BLK - 1)) >> 7
        ptok = pblk << 7
        cum_ptok = plsc.cumsum(ptok)
        pstart = cum_ptok - ptok
        cum_blk = plsc.cumsum(pblk)
        base_vec = pstart + prior
        base_splats = [splat_sum(jnp.where(iota == e, base_vec, 0))
                       for e in range(NE)]

        runs = [jnp.zeros((16,), jnp.int32) for _ in range(NE)]
        poss = []
        for t in range(16):
            tok = toks_by_step[t]
            pos = jnp.zeros((16,), jnp.int32)
            for e in range(NE):
                m = tok == e
                pos = jnp.where(m, base_splats[e] + prefs[e] + runs[e], pos)
                runs[e] = runs[e] + jnp.where(m, 1, 0)
            poss.append(pos)
            plsc.store_scatter(pos1_v, [iota * 16 + t], pos)
            tok1_v[pl.ds(t * 16, 16)] = base_tok + iota * 16 + t
        pltpu.sync_copy(pos1_v, pos_hbm.at[pl.ds(base_tok, _RT)])
        for t in range(16):
            pltpu.sync_copy(tok1_v.at[pl.ds(t * 16, 16)],
                            slot_sh.at[poss[t]])
        plsc.subcore_barrier()

        pltpu.sync_copy(slot_sh.at[pl.ds(w * _STRIPE, _STRIPE)], stripe_v)
        for c in range(_STRIPE // 16):
            st = stripe_v[pl.ds(c * 16, 16)]
            slot_pos = w * _STRIPE + c * 16 + iota
            sidx_v[pl.ds(c * 16, 16)] = jnp.where(st < 0, TOK + slot_pos, st)
        pltpu.sync_copy(sidx_v, sidx_hbm.at[pl.ds(w * _STRIPE, _STRIPE)])

        @pl.when(w == 0)
        def _():
            cb_splats = [splat_sum(jnp.where(iota == e, cum_blk, 0))
                         for e in range(NE)]
            for c in range(3):
                j = c * 16 + iota
                be = jnp.zeros((16,), jnp.int32)
                for e in range(NE):
                    be = be + jnp.where(j >= cb_splats[e], 1, 0)
                be_v[pl.ds(c * 16, 16)] = jnp.minimum(be, NE - 1)
            pltpu.sync_copy(be_v, be_hbm)

    return k(sel)


def kernel(state, W1, b1, W2, b2, W3, b3, EW1, Eb1, EW2, Eb2, EW3, Eb3):
    gumbel = jax.random.gumbel(jax.random.key(42), (TOK, NE), jnp.float32)
    probs, sel = _router(state, W1, b1, W2, b2, W3, b3, gumbel)
    pos, scatter_idx, block_expert = _sc_routing(sel)
    xg = _sc_shuffle_in(state, pos)
    y = _experts(block_expert, xg, EW1, Eb1, EW2, Eb2, EW3, Eb3)
    out = _sc_scatter(y, scatter_idx)
    action_mean = out[:TOK, :ADIM]
    action_std = out[:TOK, ADIM:]
    return probs, sel, action_mean, action_std
